# baseline (speedup 1.0000x reference)
import functools

import jax
import jax.numpy as jnp
from jax import lax
from jax.experimental import pallas as pl
from jax.experimental.pallas import tpu as pltpu
from jax.experimental.pallas import tpu_sc as plsc

N = 10000
NP = 10240
E = 320000
D = 128

NC = 2
NS = 16
NW = NC * NS
EW = E // NW
NPT = NP // NS


S0 = 166400
S1 = 153600

BE = 2560
BN = 512


def _mesh():
  return plsc.VectorSubcoreMesh(core_axis_name="c", subcore_axis_name="s",
                                num_cores=NC, num_subcores=NS)


@functools.cache
def _make_dist():
  EA = S0 // NW
  EB = S1 // NW

  @functools.partial(
      pl.kernel,
      out_type=(jax.ShapeDtypeStruct((NW, EA), jnp.float32),
                jax.ShapeDtypeStruct((NW, EB), jnp.float32)),
      mesh=_mesh(),
      compiler_params=pltpu.CompilerParams(needs_layout_passes=False),
      scratch_types=[
          pltpu.VMEM((NP * 4,), jnp.float32),
          pltpu.VMEM((EA,), jnp.int32),
          pltpu.VMEM((EA,), jnp.int32),
          pltpu.VMEM((EA,), jnp.float32),
          pltpu.VMEM((EB,), jnp.int32),
          pltpu.VMEM((EB,), jnp.int32),
          pltpu.VMEM((EB,), jnp.float32),
      ],
  )
  def distk(xp_hbm, esta_hbm, eenda_hbm, estb_hbm, eendb_hbm,
            outa_hbm, outb_hbm, xp_v, i1a, i2a, da, i1b, i2b, db):
    wid = lax.axis_index("c") * NS + lax.axis_index("s")
    pltpu.sync_copy(xp_hbm, xp_v)
    pltpu.sync_copy(esta_hbm.at[wid], i1a)
    pltpu.sync_copy(eenda_hbm.at[wid], i2a)
    pltpu.sync_copy(estb_hbm.at[wid], i1b)
    pltpu.sync_copy(eendb_hbm.at[wid], i2b)

    def make_group(i1_v, i2_v, d_v):
      def group(g, carry):
        rs = i1_v[pl.ds(g * 16, 16)] * 4
        re = i2_v[pl.ds(g * 16, 16)] * 4
        acc = jnp.zeros((16,), jnp.float32)
        for c in range(3):
          dx = (plsc.load_gather(xp_v, [rs + c]) -
                plsc.load_gather(xp_v, [re + c]))
          acc = acc + dx * dx
        bits = plsc.bitcast(acc, jnp.int32)
        y = plsc.bitcast((bits >> 1) + 0x1FBD1DF5, jnp.float32)
        y = 0.5 * (y + acc / y)
        y = 0.5 * (y + acc / y)
        y = 0.5 * (y + acc / y)
        d_v[pl.ds(g * 16, 16)] = y
        return carry
      return group

    lax.fori_loop(0, EA // 16, make_group(i1a, i2a, da), 0)
    lax.fori_loop(0, EB // 16, make_group(i1b, i2b, db), 0)
    pltpu.sync_copy(da, outa_hbm.at[wid])
    pltpu.sync_copy(db, outb_hbm.at[wid])

  return distk


@functools.cache
def _make_gather(S, CE):
  EWS = S // NW
  GNIT = EWS // CE
  assert GNIT % 2 == 0
  CW = CE

  @functools.partial(
      pl.kernel,
      out_type=jax.ShapeDtypeStruct((S, D), jnp.float32),
      mesh=_mesh(),
      compiler_params=pltpu.CompilerParams(needs_layout_passes=False),
      scratch_types=[
          pltpu.VMEM((2, 1, CW), jnp.int32),
          pltpu.VMEM((2, 1, CW), jnp.int32),
          pltpu.VMEM((2, CE, D), jnp.float32),
          pltpu.VMEM((2, CE, D), jnp.float32),
          pltpu.VMEM((EWS,), jnp.float32),
          pltpu.VMEM((D,), jnp.float32),
          pltpu.SemaphoreType.DMA((2,)),
          pltpu.SemaphoreType.DMA((2,)),
          pltpu.SemaphoreType.DMA((2,)),
      ],
  )
  def gather(t1_hbm, t2_hbm, est_hbm, eend_hbm, dist_hbm, w1c_hbm, out_hbm,
             idx1_v, idx2_v, r1_v, r2_v, dist_v, w1c_v, sem1, sem2, semw):
    wid = lax.axis_index("c") * NS + lax.axis_index("s")
    blk0 = wid * GNIT
    e0 = wid * EWS
    pltpu.sync_copy(dist_hbm.at[wid], dist_v)
    pltpu.sync_copy(w1c_hbm, w1c_v)

    def fire(k, b):
      pltpu.sync_copy(est_hbm.at[blk0 + k], idx1_v.at[b])
      pltpu.sync_copy(eend_hbm.at[blk0 + k], idx2_v.at[b])
      pltpu.async_copy(t1_hbm.at[idx1_v.at[b, 0]], r1_v.at[b], sem1.at[b])
      pltpu.async_copy(t2_hbm.at[idx2_v.at[b, 0]], r2_v.at[b], sem2.at[b])

    fire(0, 0)

    def body(k2, carry):
      for b in (0, 1):
        k = 2 * k2 + b
        nb = 1 - b

        @pl.when(k > 0)
        def _():
          pltpu.make_async_copy(r1_v.at[nb], out_hbm.at[pl.ds(0, CE)],
                                semw.at[nb]).wait()

        @pl.when(k + 1 < GNIT)
        def _():
          fire(k + 1, nb)

        pltpu.make_async_copy(t1_hbm.at[pl.ds(0, CE)], r1_v.at[b],
                              sem1.at[b]).wait()
        pltpu.make_async_copy(t2_hbm.at[pl.ds(0, CE)], r2_v.at[b],
                              sem2.at[b]).wait()

        w1segs = [w1c_v[pl.ds(s * 16, 16)] for s in range(D // 16)]

        def add_row(i, c2):
          dscale = plsc.load_gather(
              dist_v, [jnp.full((16,), k * CE + i, jnp.int32)])
          for s in range(D // 16):
            sl = (b, i, pl.ds(s * 16, 16))
            r1_v[sl] = r1_v[sl] + r2_v[sl] + dscale * w1segs[s]
          return c2
        lax.fori_loop(0, CE, add_row, 0)

        pltpu.async_copy(r1_v.at[b], out_hbm.at[pl.ds(e0 + k * CE, CE)],
                         semw.at[b])
      return carry

    lax.fori_loop(0, GNIT // 2, body, 0)
    pltpu.make_async_copy(r1_v.at[1], out_hbm.at[pl.ds(0, CE)],
                          semw.at[1]).wait()

  return gather


@functools.cache
def _make_scatter(S, SCE):
  EWS = S // NW
  TNIT = EWS // SCE
  SKB = 1
  SCW = SCE

  @functools.partial(
      pl.kernel,
      out_type=jax.ShapeDtypeStruct((NC, NP, D), jnp.float32),
      mesh=_mesh(),
      scratch_types=[
          pltpu.VMEM((2, SKB, SCW), jnp.int32),
          pltpu.VMEM((2, SCE, D), jnp.float32),
          pltpu.VMEM_SHARED((NP, D), jnp.float32),
          pltpu.SemaphoreType.DMA((2,)),
      ],
  )
  def scatter(src_hbm, est_hbm, zeros_hbm, out_hbm, idx_v, buf_v, acc_sh,
              sems):
    cid = lax.axis_index("c")
    sid = lax.axis_index("s")
    wid = cid * NS + sid
    blk0 = wid * TNIT
    e0 = wid * EWS

    pltpu.sync_copy(zeros_hbm.at[pl.ds(sid * NPT, NPT)],
                    acc_sh.at[pl.ds(sid * NPT, NPT)])
    plsc.subcore_barrier()

    def fire(k, b):
      pltpu.sync_copy(est_hbm.at[blk0 + k], idx_v.at[b])
      pltpu.async_copy(src_hbm.at[pl.ds(e0 + k * SCE, SCE)], buf_v.at[b],
                       sems.at[b])

    def consume(k, b):
      pltpu.make_async_copy(src_hbm.at[pl.ds(0, SCE)], buf_v.at[b],
                            sems.at[b]).wait()
      for j in range(SKB):
        pltpu.sync_copy(buf_v.at[b, pl.ds(j * SCW, SCW)],
                        acc_sh.at[idx_v.at[b, j]], add=True)

    fire(0, 0)

    def body(k2, carry):
      for b in (0, 1):
        k = 2 * k2 + b

        @pl.when(k + 1 < TNIT)
        def _():
          fire(k + 1, 1 - b)

        consume(k, b)
      return carry

    lax.fori_loop(0, TNIT // 2, body, 0)
    if TNIT % 2:
      consume(TNIT - 1, 0)

    plsc.subcore_barrier()
    pltpu.sync_copy(acc_sh.at[pl.ds(sid * NPT, NPT)],
                    out_hbm.at[cid, pl.ds(sid * NPT, NPT)])

  return scatter


def _silu(v):
  return v * jax.nn.sigmoid(v)


def _dot(a, b):
  return jnp.dot(a, b, preferred_element_type=jnp.float32)


def _prep_body(h_ref, we_ref, be_ref, w1a_ref, w1b_ref, b1_ref,
               h0_ref, t1_ref, t2_ref):
  h0 = _dot(h_ref[...], we_ref[...]) + be_ref[...]
  h0_ref[...] = h0
  t1_ref[...] = _dot(h0, w1a_ref[...]) + b1_ref[...]
  t2_ref[...] = _dot(h0, w1b_ref[...])


def _edge_body(g_ref, w2_ref, b2_ref, winf_ref, binf_ref, src_ref):
  m = _silu(g_ref[...])
  m2 = _silu(_dot(m, w2_ref[...]) + b2_ref[...])
  logit = jnp.sum(m2 * winf_ref[...], axis=1, keepdims=True) + binf_ref[0, 0]
  src_ref[...] = jax.nn.sigmoid(logit) * m2


def _node0_body(h_ref, ma_ref, mb_ref, wa_ref, wb_ref, b1_ref, w2_ref, b2_ref,
                w1an_ref, w1bn_ref, b1n_ref, hn_ref, t1_ref, t2_ref):
  mi = ma_ref[0] + ma_ref[1] + mb_ref[0] + mb_ref[1]
  u = _silu(_dot(h_ref[...], wa_ref[...]) + _dot(mi, wb_ref[...]) + b1_ref[...])
  hn = h_ref[...] + _dot(u, w2_ref[...]) + b2_ref[...]
  hn_ref[...] = hn
  t1_ref[...] = _dot(hn, w1an_ref[...]) + b1n_ref[...]
  t2_ref[...] = _dot(hn, w1bn_ref[...])


def _node1_body(h_ref, ma_ref, mb_ref, wa_ref, wb_ref, b1_ref, w2_ref, b2_ref,
                hn_ref):
  mi = ma_ref[0] + ma_ref[1] + mb_ref[0] + mb_ref[1]
  u = _silu(_dot(h_ref[...], wa_ref[...]) + _dot(mi, wb_ref[...]) + b1_ref[...])
  hn_ref[...] = h_ref[...] + _dot(u, w2_ref[...]) + b2_ref[...]


def _wspec(shape):
  return pl.BlockSpec(shape, lambda i: (0,) * len(shape))


def _rspec(rows, cols):
  return pl.BlockSpec((rows, cols), lambda i: (i, 0))


_NGRID = NP // BN


def _pspec():
  return pl.BlockSpec((2, BN, D), lambda i: (0, i, 0))


_prep = pl.pallas_call(
    _prep_body,
    grid=(_NGRID,),
    in_specs=[_rspec(BN, D), _wspec((D, D)), _wspec((1, D)), _wspec((D, D)),
              _wspec((D, D)), _wspec((1, D))],
    out_specs=[_rspec(BN, D)] * 3,
    out_shape=[jax.ShapeDtypeStruct((NP, D), jnp.float32)] * 3,
)

@functools.cache
def _make_edge(S, off):
  blk_off = off // BE
  return pl.pallas_call(
      _edge_body,
      grid=(S // BE,),
      in_specs=[_rspec(BE, D), _wspec((D, D)), _wspec((1, D)), _wspec((1, D)),
                pl.BlockSpec(memory_space=pltpu.SMEM)],
      out_specs=_rspec(BE, D),
      out_shape=jax.ShapeDtypeStruct((S, D), jnp.float32),
  )

_node0 = pl.pallas_call(
    _node0_body,
    grid=(_NGRID,),
    in_specs=[_rspec(BN, D), _pspec(), _pspec()] + [_wspec((D, D)),
              _wspec((D, D)), _wspec((1, D)), _wspec((D, D)), _wspec((1, D)),
              _wspec((D, D)), _wspec((D, D)), _wspec((1, D))],
    out_specs=[_rspec(BN, D)] * 3,
    out_shape=[jax.ShapeDtypeStruct((NP, D), jnp.float32)] * 3,
)

_node1 = pl.pallas_call(
    _node1_body,
    grid=(_NGRID,),
    in_specs=[_rspec(BN, D), _pspec(), _pspec()] + [_wspec((D, D)),
              _wspec((D, D)), _wspec((1, D)), _wspec((D, D)), _wspec((1, D))],
    out_specs=_rspec(BN, D),
    out_shape=jax.ShapeDtypeStruct((NP, D), jnp.float32),
)


def kernel(h, x, e, W_emb, b_emb, fe_w1, fe_b1, fe_w2, fe_b2,
           finf_w, finf_b, fh_w1, fh_b1, fh_w2, fh_b2):
  f32 = jnp.float32
  ei = e.astype(jnp.int32)
  SCEA, SCEB = 104, 96
  sa = ei[0, :S0].reshape(S0 // SCEA, 1, SCEA)
  sb = ei[0, S0:].reshape(S1 // SCEB, 1, SCEB)

  ga = (ei[0, :S0].reshape(S0 // SCEA, 1, SCEA),
        ei[1, :S0].reshape(S0 // SCEA, 1, SCEA))
  gb = (ei[0, S0:].reshape(S1 // SCEB, 1, SCEB),
        ei[1, S0:].reshape(S1 // SCEB, 1, SCEB))
  wa = (ei[0, :S0].reshape(NW, S0 // NW), ei[1, :S0].reshape(NW, S0 // NW))
  wb = (ei[0, S0:].reshape(NW, S1 // NW), ei[1, S0:].reshape(NW, S1 // NW))

  hp = jnp.pad(h.astype(f32), ((0, NP - N), (0, 0)))
  xp = jnp.pad(x.astype(f32), ((0, NP - N), (0, 1))).reshape(-1)
  zeros_np = jnp.zeros((NP, D), f32)

  r = lambda a: a.reshape(1, D)
  sc = lambda a: a.reshape(1, 1)

  dist_a, dist_b = _make_dist()(xp, *wa, *wb)
  gather_a = _make_gather(S0, SCEA)
  gather_b = _make_gather(S1, SCEB)
  scatter_a = _make_scatter(S0, SCEA)
  scatter_b = _make_scatter(S1, SCEB)
  edge_a = _make_edge(S0, 0)
  edge_b = _make_edge(S1, S0)

  h0, t1, t2 = _prep(hp, W_emb, r(b_emb), fe_w1[0, :D], fe_w1[0, D:2 * D],
                     r(fe_b1[0]))

  def layer(hcur, t1c, t2c, l, last):
    w1c = fe_w1[l, 2 * D]
    ew = (fe_w2[l], r(fe_b2[l]), r(finf_w[l]), sc(finf_b[l]))
    g_a = gather_a(t1c, t2c, *ga, dist_a, w1c)
    g_b = gather_b(t1c, t2c, *gb, dist_b, w1c)
    src_a = edge_a(g_a, *ew)
    src_b = edge_b(g_b, *ew)
    p_a = scatter_a(src_a, sa, zeros_np)
    p_b = scatter_b(src_b, sb, zeros_np)
    nw = (fh_w1[l, :D], fh_w1[l, D:], r(fh_b1[l]), fh_w2[l], r(fh_b2[l]))
    if last:
      return _node1(hcur, p_a, p_b, *nw)
    return _node0(hcur, p_a, p_b, *nw,
                  fe_w1[l + 1, :D], fe_w1[l + 1, D:2 * D], r(fe_b1[l + 1]))

  h1, t1b, t2b = layer(h0, t1, t2, 0, False)
  h2 = layer(h1, t1b, t2b, 1, True)

  return (h2[:N], e)


# speedup vs baseline: 1.0398x; 1.0398x over previous
import functools

import jax
import jax.numpy as jnp
from jax import lax
from jax.experimental import pallas as pl
from jax.experimental.pallas import tpu as pltpu
from jax.experimental.pallas import tpu_sc as plsc

N = 10000
NP = 10240
E = 320000
D = 128

NC = 2
NS = 16
NW = NC * NS
EW = E // NW
CW = 40
KB = 5
CE = KB * CW
NPT = NP // NS


S0 = 166400
S1 = 153600

BE = 2560
BN = 512


def _mesh():
  return plsc.VectorSubcoreMesh(core_axis_name="c", subcore_axis_name="s",
                                num_cores=NC, num_subcores=NS)


@functools.cache
def _make_dist():
  EA = S0 // NW
  EB = S1 // NW

  @functools.partial(
      pl.kernel,
      out_type=(jax.ShapeDtypeStruct((NW, EA), jnp.float32),
                jax.ShapeDtypeStruct((NW, EB), jnp.float32)),
      mesh=_mesh(),
      compiler_params=pltpu.CompilerParams(needs_layout_passes=False),
      scratch_types=[
          pltpu.VMEM((NP * 4,), jnp.float32),
          pltpu.VMEM((EA,), jnp.int32),
          pltpu.VMEM((EA,), jnp.int32),
          pltpu.VMEM((EA,), jnp.float32),
          pltpu.VMEM((EB,), jnp.int32),
          pltpu.VMEM((EB,), jnp.int32),
          pltpu.VMEM((EB,), jnp.float32),
      ],
  )
  def distk(xp_hbm, esta_hbm, eenda_hbm, estb_hbm, eendb_hbm,
            outa_hbm, outb_hbm, xp_v, i1a, i2a, da, i1b, i2b, db):
    wid = lax.axis_index("c") * NS + lax.axis_index("s")
    pltpu.sync_copy(xp_hbm, xp_v)
    pltpu.sync_copy(esta_hbm.at[wid], i1a)
    pltpu.sync_copy(eenda_hbm.at[wid], i2a)
    pltpu.sync_copy(estb_hbm.at[wid], i1b)
    pltpu.sync_copy(eendb_hbm.at[wid], i2b)

    def make_group(i1_v, i2_v, d_v):
      def group(g, carry):
        rs = i1_v[pl.ds(g * 16, 16)] * 4
        re = i2_v[pl.ds(g * 16, 16)] * 4
        acc = jnp.zeros((16,), jnp.float32)
        for c in range(3):
          dx = (plsc.load_gather(xp_v, [rs + c]) -
                plsc.load_gather(xp_v, [re + c]))
          acc = acc + dx * dx
        bits = plsc.bitcast(acc, jnp.int32)
        y = plsc.bitcast((bits >> 1) + 0x1FBD1DF5, jnp.float32)
        y = 0.5 * (y + acc / y)
        y = 0.5 * (y + acc / y)
        y = 0.5 * (y + acc / y)
        d_v[pl.ds(g * 16, 16)] = y
        return carry
      return group

    lax.fori_loop(0, EA // 16, make_group(i1a, i2a, da), 0)
    lax.fori_loop(0, EB // 16, make_group(i1b, i2b, db), 0)
    pltpu.sync_copy(da, outa_hbm.at[wid])
    pltpu.sync_copy(db, outb_hbm.at[wid])

  return distk


@functools.cache
def _make_gather(S):
  EWS = S // NW
  GNIT = EWS // CE
  assert GNIT % 2 == 0

  @functools.partial(
      pl.kernel,
      out_type=jax.ShapeDtypeStruct((S, D), jnp.float32),
      mesh=_mesh(),
      compiler_params=pltpu.CompilerParams(needs_layout_passes=False),
      scratch_types=[
          pltpu.VMEM((2, KB, CW), jnp.int32),
          pltpu.VMEM((2, KB, CW), jnp.int32),
          pltpu.VMEM((2, CE, D), jnp.float32),
          pltpu.VMEM((2, CE, D), jnp.float32),
          pltpu.VMEM((EWS,), jnp.float32),
          pltpu.VMEM((D,), jnp.float32),
          pltpu.SemaphoreType.DMA((2,)),
          pltpu.SemaphoreType.DMA((2,)),
          pltpu.SemaphoreType.DMA((2,)),
      ],
  )
  def gather(t1_hbm, t2_hbm, est_hbm, eend_hbm, dist_hbm, w1c_hbm, out_hbm,
             idx1_v, idx2_v, r1_v, r2_v, dist_v, w1c_v, sem1, sem2, semw):
    wid = lax.axis_index("c") * NS + lax.axis_index("s")
    blk0 = wid * GNIT
    e0 = wid * EWS
    pltpu.sync_copy(dist_hbm.at[wid], dist_v)
    pltpu.sync_copy(w1c_hbm, w1c_v)

    def fire(k, b):
      pltpu.sync_copy(est_hbm.at[blk0 + k], idx1_v.at[b])
      pltpu.sync_copy(eend_hbm.at[blk0 + k], idx2_v.at[b])
      for j in range(KB):
        pltpu.async_copy(t1_hbm.at[idx1_v.at[b, j]],
                         r1_v.at[b, pl.ds(j * CW, CW)], sem1.at[b])
        pltpu.async_copy(t2_hbm.at[idx2_v.at[b, j]],
                         r2_v.at[b, pl.ds(j * CW, CW)], sem2.at[b])

    fire(0, 0)

    def body(k2, carry):
      for b in (0, 1):
        k = 2 * k2 + b
        nb = 1 - b

        @pl.when(k > 0)
        def _():
          pltpu.make_async_copy(r1_v.at[nb], out_hbm.at[pl.ds(0, CE)],
                                semw.at[nb]).wait()

        @pl.when(k + 1 < GNIT)
        def _():
          fire(k + 1, nb)

        pltpu.make_async_copy(t1_hbm.at[pl.ds(0, CE)], r1_v.at[b],
                              sem1.at[b]).wait()
        pltpu.make_async_copy(t2_hbm.at[pl.ds(0, CE)], r2_v.at[b],
                              sem2.at[b]).wait()

        w1segs = [w1c_v[pl.ds(s * 16, 16)] for s in range(D // 16)]

        def add_row(i, c2):
          dscale = plsc.load_gather(
              dist_v, [jnp.full((16,), k * CE + i, jnp.int32)])
          for s in range(D // 16):
            sl = (b, i, pl.ds(s * 16, 16))
            r1_v[sl] = r1_v[sl] + r2_v[sl] + dscale * w1segs[s]
          return c2
        lax.fori_loop(0, CE, add_row, 0)

        pltpu.async_copy(r1_v.at[b], out_hbm.at[pl.ds(e0 + k * CE, CE)],
                         semw.at[b])
      return carry

    lax.fori_loop(0, GNIT // 2, body, 0)
    pltpu.make_async_copy(r1_v.at[1], out_hbm.at[pl.ds(0, CE)],
                          semw.at[1]).wait()

  return gather


@functools.cache
def _make_scatter(S, SCE):
  EWS = S // NW
  TNIT = EWS // SCE
  SKB = 1
  SCW = SCE

  @functools.partial(
      pl.kernel,
      out_type=jax.ShapeDtypeStruct((NC, NP, D), jnp.float32),
      mesh=_mesh(),
      scratch_types=[
          pltpu.VMEM((2, SKB, SCW), jnp.int32),
          pltpu.VMEM((2, SCE, D), jnp.float32),
          pltpu.VMEM_SHARED((NP, D), jnp.float32),
          pltpu.SemaphoreType.DMA((2,)),
      ],
  )
  def scatter(src_hbm, est_hbm, zeros_hbm, out_hbm, idx_v, buf_v, acc_sh,
              sems):
    cid = lax.axis_index("c")
    sid = lax.axis_index("s")
    wid = cid * NS + sid
    blk0 = wid * TNIT
    e0 = wid * EWS

    pltpu.sync_copy(zeros_hbm.at[pl.ds(sid * NPT, NPT)],
                    acc_sh.at[pl.ds(sid * NPT, NPT)])
    plsc.subcore_barrier()

    def fire(k, b):
      pltpu.sync_copy(est_hbm.at[blk0 + k], idx_v.at[b])
      pltpu.async_copy(src_hbm.at[pl.ds(e0 + k * SCE, SCE)], buf_v.at[b],
                       sems.at[b])

    def consume(k, b):
      pltpu.make_async_copy(src_hbm.at[pl.ds(0, SCE)], buf_v.at[b],
                            sems.at[b]).wait()
      for j in range(SKB):
        pltpu.sync_copy(buf_v.at[b, pl.ds(j * SCW, SCW)],
                        acc_sh.at[idx_v.at[b, j]], add=True)

    fire(0, 0)

    def body(k2, carry):
      for b in (0, 1):
        k = 2 * k2 + b

        @pl.when(k + 1 < TNIT)
        def _():
          fire(k + 1, 1 - b)

        consume(k, b)
      return carry

    lax.fori_loop(0, TNIT // 2, body, 0)
    if TNIT % 2:
      consume(TNIT - 1, 0)

    plsc.subcore_barrier()
    pltpu.sync_copy(acc_sh.at[pl.ds(sid * NPT, NPT)],
                    out_hbm.at[cid, pl.ds(sid * NPT, NPT)])

  return scatter


def _silu(v):
  return v * jax.nn.sigmoid(v)


def _dot(a, b):
  return jnp.dot(a, b, preferred_element_type=jnp.float32)


def _prep_body(h_ref, we_ref, be_ref, w1a_ref, w1b_ref, b1_ref,
               h0_ref, t1_ref, t2_ref):
  h0 = _dot(h_ref[...], we_ref[...]) + be_ref[...]
  h0_ref[...] = h0
  t1_ref[...] = _dot(h0, w1a_ref[...]) + b1_ref[...]
  t2_ref[...] = _dot(h0, w1b_ref[...])


def _edge_body(g_ref, w2_ref, b2_ref, winf_ref, binf_ref, src_ref):
  m = _silu(g_ref[...])
  m2 = _silu(_dot(m, w2_ref[...]) + b2_ref[...])
  logit = jnp.sum(m2 * winf_ref[...], axis=1, keepdims=True) + binf_ref[0, 0]
  src_ref[...] = jax.nn.sigmoid(logit) * m2


def _node0_body(h_ref, ma_ref, mb_ref, wa_ref, wb_ref, b1_ref, w2_ref, b2_ref,
                w1an_ref, w1bn_ref, b1n_ref, hn_ref, t1_ref, t2_ref):
  mi = ma_ref[0] + ma_ref[1] + mb_ref[0] + mb_ref[1]
  u = _silu(_dot(h_ref[...], wa_ref[...]) + _dot(mi, wb_ref[...]) + b1_ref[...])
  hn = h_ref[...] + _dot(u, w2_ref[...]) + b2_ref[...]
  hn_ref[...] = hn
  t1_ref[...] = _dot(hn, w1an_ref[...]) + b1n_ref[...]
  t2_ref[...] = _dot(hn, w1bn_ref[...])


def _node1_body(h_ref, ma_ref, mb_ref, wa_ref, wb_ref, b1_ref, w2_ref, b2_ref,
                hn_ref):
  mi = ma_ref[0] + ma_ref[1] + mb_ref[0] + mb_ref[1]
  u = _silu(_dot(h_ref[...], wa_ref[...]) + _dot(mi, wb_ref[...]) + b1_ref[...])
  hn_ref[...] = h_ref[...] + _dot(u, w2_ref[...]) + b2_ref[...]


def _wspec(shape):
  return pl.BlockSpec(shape, lambda i: (0,) * len(shape))


def _rspec(rows, cols):
  return pl.BlockSpec((rows, cols), lambda i: (i, 0))


_NGRID = NP // BN


def _pspec():
  return pl.BlockSpec((2, BN, D), lambda i: (0, i, 0))


_prep = pl.pallas_call(
    _prep_body,
    grid=(_NGRID,),
    in_specs=[_rspec(BN, D), _wspec((D, D)), _wspec((1, D)), _wspec((D, D)),
              _wspec((D, D)), _wspec((1, D))],
    out_specs=[_rspec(BN, D)] * 3,
    out_shape=[jax.ShapeDtypeStruct((NP, D), jnp.float32)] * 3,
)

@functools.cache
def _make_edge(S, off):
  blk_off = off // BE
  return pl.pallas_call(
      _edge_body,
      grid=(S // BE,),
      in_specs=[_rspec(BE, D), _wspec((D, D)), _wspec((1, D)), _wspec((1, D)),
                pl.BlockSpec(memory_space=pltpu.SMEM)],
      out_specs=_rspec(BE, D),
      out_shape=jax.ShapeDtypeStruct((S, D), jnp.float32),
  )

_node0 = pl.pallas_call(
    _node0_body,
    grid=(_NGRID,),
    in_specs=[_rspec(BN, D), _pspec(), _pspec()] + [_wspec((D, D)),
              _wspec((D, D)), _wspec((1, D)), _wspec((D, D)), _wspec((1, D)),
              _wspec((D, D)), _wspec((D, D)), _wspec((1, D))],
    out_specs=[_rspec(BN, D)] * 3,
    out_shape=[jax.ShapeDtypeStruct((NP, D), jnp.float32)] * 3,
)

_node1 = pl.pallas_call(
    _node1_body,
    grid=(_NGRID,),
    in_specs=[_rspec(BN, D), _pspec(), _pspec()] + [_wspec((D, D)),
              _wspec((D, D)), _wspec((1, D)), _wspec((D, D)), _wspec((1, D))],
    out_specs=_rspec(BN, D),
    out_shape=jax.ShapeDtypeStruct((NP, D), jnp.float32),
)


def kernel(h, x, e, W_emb, b_emb, fe_w1, fe_b1, fe_w2, fe_b2,
           finf_w, finf_b, fh_w1, fh_b1, fh_w2, fh_b2):
  f32 = jnp.float32
  ei = e.astype(jnp.int32)
  est3d = ei[0].reshape(E // CE, KB, CW)
  eend3d = ei[1].reshape(E // CE, KB, CW)
  SCEA, SCEB = 104, 96
  sa = ei[0, :S0].reshape(S0 // SCEA, 1, SCEA)
  sb = ei[0, S0:].reshape(S1 // SCEB, 1, SCEB)

  ga = (est3d[:S0 // CE], eend3d[:S0 // CE])
  gb = (est3d[S0 // CE:], eend3d[S0 // CE:])
  wa = (ei[0, :S0].reshape(NW, S0 // NW), ei[1, :S0].reshape(NW, S0 // NW))
  wb = (ei[0, S0:].reshape(NW, S1 // NW), ei[1, S0:].reshape(NW, S1 // NW))

  hp = jnp.pad(h.astype(f32), ((0, NP - N), (0, 0)))
  xp = jnp.pad(x.astype(f32), ((0, NP - N), (0, 1))).reshape(-1)
  zeros_np = jnp.zeros((NP, D), f32)

  r = lambda a: a.reshape(1, D)
  sc = lambda a: a.reshape(1, 1)

  dist_a, dist_b = _make_dist()(xp, *wa, *wb)
  gather_a = _make_gather(S0)
  gather_b = _make_gather(S1)
  scatter_a = _make_scatter(S0, SCEA)
  scatter_b = _make_scatter(S1, SCEB)
  edge_a = _make_edge(S0, 0)
  edge_b = _make_edge(S1, S0)

  h0, t1, t2 = _prep(hp, W_emb, r(b_emb), fe_w1[0, :D], fe_w1[0, D:2 * D],
                     r(fe_b1[0]))

  def layer(hcur, t1c, t2c, l, last):
    w1c = fe_w1[l, 2 * D]
    ew = (fe_w2[l], r(fe_b2[l]), r(finf_w[l]), sc(finf_b[l]))
    g_a = gather_a(t1c, t2c, *ga, dist_a, w1c)
    g_b = gather_b(t1c, t2c, *gb, dist_b, w1c)
    src_a = edge_a(g_a, *ew)
    src_b = edge_b(g_b, *ew)
    p_a = scatter_a(src_a, sa, zeros_np)
    p_b = scatter_b(src_b, sb, zeros_np)
    nw = (fh_w1[l, :D], fh_w1[l, D:], r(fh_b1[l]), fh_w2[l], r(fh_b2[l]))
    if last:
      return _node1(hcur, p_a, p_b, *nw)
    return _node0(hcur, p_a, p_b, *nw,
                  fe_w1[l + 1, :D], fe_w1[l + 1, D:2 * D], r(fe_b1[l + 1]))

  h1, t1b, t2b = layer(h0, t1, t2, 0, False)
  h2 = layer(h1, t1b, t2b, 1, True)

  return (h2[:N], e)
